# Initial kernel scaffold; baseline (speedup 1.0000x reference)
import jax
import jax.numpy as jnp
from jax.experimental import pallas as pl


def kernel(x, emb_table, pos_table):
    raise NotImplementedError("write your pallas kernel here")


# speedup vs baseline: 8.4526x; 8.4526x over previous
import functools

import jax
import jax.numpy as jnp
from jax import lax
from jax.experimental import pallas as pl
from jax.experimental.pallas import tpu as pltpu
from jax.experimental.pallas import tpu_sc as plsc

D_MODEL = 128
CHUNK = 128


def _fuse_body(emb_ref, pos_ref, out_ref):
    out_ref[...] = emb_ref[...] + pos_ref[...]


def _fuse_tables(emb_head, pos_table):
    return pl.pallas_call(
        _fuse_body,
        out_shape=jax.ShapeDtypeStruct(pos_table.shape, jnp.float32),
    )(emb_head, pos_table)


@functools.lru_cache(maxsize=None)
def _make_gather(nb, d):
    info = plsc.get_sparse_core_info()
    nc, ns = info.num_cores, info.num_subcores
    nw = nc * ns
    b_per_w = nb // nw
    n_chunks = b_per_w // CHUNK
    mesh = plsc.VectorSubcoreMesh(core_axis_name="c", subcore_axis_name="s")

    @functools.partial(
        pl.kernel,
        mesh=mesh,
        out_type=jax.ShapeDtypeStruct((nb, d), jnp.float32),
        scratch_types=[
            pltpu.VMEM((n_chunks, CHUNK), jnp.int32),
            pltpu.VMEM((CHUNK, d), jnp.float32),
            pltpu.SemaphoreType.DMA,
        ],
    )
    def gather(table_hbm, idx_hbm, out_hbm, idx_v, rows_v, sem):
        wid = lax.axis_index("s") * nc + lax.axis_index("c")
        pltpu.sync_copy(idx_hbm.at[wid], idx_v)
        base = wid * b_per_w

        def body(g, carry):
            pltpu.async_copy(table_hbm.at[idx_v.at[g]], rows_v, sem).wait()
            pltpu.sync_copy(rows_v, out_hbm.at[pl.ds(base + g * CHUNK, CHUNK)])
            return carry

        lax.fori_loop(0, n_chunks, body, 0)

    return gather


def kernel(x, emb_table, pos_table):
    b, s = x.shape
    nb = b * s
    t_rows = pos_table.shape[0]
    fused = _fuse_tables(emb_table[:t_rows], pos_table)
    info = plsc.get_sparse_core_info()
    nw = info.num_cores * info.num_subcores
    b_per_w = nb // nw
    idx = x.reshape(nw, b_per_w // CHUNK, CHUNK).astype(jnp.int32)
    out = _make_gather(nb, D_MODEL)(fused, idx)
    return out.reshape(b, s, D_MODEL)


# speedup vs baseline: 8.4978x; 1.0054x over previous
import functools

import jax
import jax.numpy as jnp
from jax import lax
from jax.experimental import pallas as pl
from jax.experimental.pallas import tpu as pltpu
from jax.experimental.pallas import tpu_sc as plsc

D_MODEL = 128
CHUNK = 128


def _fuse_body(emb_ref, pos_ref, out_ref):
    out_ref[...] = emb_ref[...] + pos_ref[...]


def _fuse_tables(emb_head, pos_table):
    return pl.pallas_call(
        _fuse_body,
        out_shape=jax.ShapeDtypeStruct(pos_table.shape, jnp.float32),
    )(emb_head, pos_table)


NBUF = 4


@functools.lru_cache(maxsize=None)
def _make_gather(nb, d):
    info = plsc.get_sparse_core_info()
    nc, ns = info.num_cores, info.num_subcores
    nw = nc * ns
    b_per_w = nb // nw
    n_chunks = b_per_w // CHUNK
    n_rounds = n_chunks // NBUF
    mesh = plsc.VectorSubcoreMesh(core_axis_name="c", subcore_axis_name="s")

    @functools.partial(
        pl.kernel,
        mesh=mesh,
        out_type=jax.ShapeDtypeStruct((nb, d), jnp.float32),
        scratch_types=[
            pltpu.VMEM((n_chunks, CHUNK), jnp.int32),
            pltpu.VMEM((NBUF, CHUNK, d), jnp.float32),
        ]
        + [pltpu.SemaphoreType.DMA] * (2 * NBUF),
    )
    def gather(table_hbm, idx_hbm, out_hbm, idx_v, rows_v, *sems):
        gsems, ssems = sems[:NBUF], sems[NBUF:]
        wid = lax.axis_index("s") * nc + lax.axis_index("c")
        pltpu.sync_copy(idx_hbm.at[wid], idx_v)
        base = wid * b_per_w

        def g_copy(b, g):
            return pltpu.make_async_copy(
                table_hbm.at[idx_v.at[g]], rows_v.at[b], gsems[b]
            )

        def s_copy(b, g):
            return pltpu.make_async_copy(
                rows_v.at[b], out_hbm.at[pl.ds(base + g * CHUNK, CHUNK)], ssems[b]
            )

        def body(t, carry):
            g0 = t * NBUF
            for b in range(NBUF):
                @pl.when(t > 0)
                def _(b=b, g0=g0):
                    s_copy(b, g0 + b - NBUF).wait()

                g_copy(b, g0 + b).start()
            for b in range(NBUF):
                g_copy(b, g0 + b).wait()
                s_copy(b, g0 + b).start()
            return carry

        lax.fori_loop(0, n_rounds, body, 0)
        for b in range(NBUF):
            s_copy(b, n_chunks - NBUF + b).wait()

    return gather


def kernel(x, emb_table, pos_table):
    b, s = x.shape
    nb = b * s
    t_rows = pos_table.shape[0]
    fused = _fuse_tables(emb_table[:t_rows], pos_table)
    info = plsc.get_sparse_core_info()
    nw = info.num_cores * info.num_subcores
    b_per_w = nb // nw
    idx = x.reshape(nw, b_per_w // CHUNK, CHUNK).astype(jnp.int32)
    out = _make_gather(nb, D_MODEL)(fused, idx)
    return out.reshape(b, s, D_MODEL)


# speedup vs baseline: 19.7687x; 2.3263x over previous
import functools

import jax
import jax.numpy as jnp
from jax import lax
from jax.experimental import pallas as pl
from jax.experimental.pallas import tpu as pltpu
from jax.experimental.pallas import tpu_sc as plsc

D_MODEL = 128
CHUNK = 128


def _fuse_body(emb_ref, pos_ref, out_ref):
    out_ref[...] = emb_ref[...] + pos_ref[...]


def _fuse_tables(emb_head, pos_table):
    return pl.pallas_call(
        _fuse_body,
        out_shape=jax.ShapeDtypeStruct(pos_table.shape, jnp.float32),
    )(emb_head, pos_table)


NBUF = 2
TABLE_ROWS = 512


@functools.lru_cache(maxsize=None)
def _make_gather(nb, d):
    info = plsc.get_sparse_core_info()
    nc, ns = info.num_cores, info.num_subcores
    nw = nc * ns
    b_per_w = nb // nw
    n_chunks = b_per_w // CHUNK
    n_rounds = n_chunks // NBUF
    mesh = plsc.VectorSubcoreMesh(core_axis_name="c", subcore_axis_name="s")

    @functools.partial(
        pl.kernel,
        mesh=mesh,
        out_type=jax.ShapeDtypeStruct((nb, d), jnp.float32),
        scratch_types=[
            pltpu.VMEM_SHARED((TABLE_ROWS, d), jnp.float32),
            pltpu.VMEM((n_chunks, CHUNK), jnp.int32),
            pltpu.VMEM((NBUF, CHUNK, d), jnp.float32),
        ]
        + [pltpu.SemaphoreType.DMA] * (2 * NBUF),
    )
    def gather(table_hbm, idx_hbm, out_hbm, table_s, idx_v, rows_v, *sems):
        gsems, ssems = sems[:NBUF], sems[NBUF:]
        sid = lax.axis_index("s")
        wid = sid * nc + lax.axis_index("c")

        @pl.when(sid == 0)
        def _():
            pltpu.sync_copy(table_hbm, table_s)

        pltpu.sync_copy(idx_hbm.at[wid], idx_v)
        plsc.subcore_barrier()
        base = wid * b_per_w

        def g_copy(b, g):
            return pltpu.make_async_copy(
                table_s.at[idx_v.at[g]], rows_v.at[b], gsems[b]
            )

        def s_copy(b, g):
            return pltpu.make_async_copy(
                rows_v.at[b], out_hbm.at[pl.ds(base + g * CHUNK, CHUNK)], ssems[b]
            )

        def body(t, carry):
            g0 = t * NBUF
            for b in range(NBUF):
                @pl.when(t > 0)
                def _(b=b, g0=g0):
                    s_copy(b, g0 + b - NBUF).wait()

                g_copy(b, g0 + b).start()
            for b in range(NBUF):
                g_copy(b, g0 + b).wait()
                s_copy(b, g0 + b).start()
            return carry

        lax.fori_loop(0, n_rounds, body, 0)
        for b in range(NBUF):
            s_copy(b, n_chunks - NBUF + b).wait()

    return gather


def kernel(x, emb_table, pos_table):
    b, s = x.shape
    nb = b * s
    t_rows = pos_table.shape[0]
    fused = _fuse_tables(emb_table[:t_rows], pos_table)
    info = plsc.get_sparse_core_info()
    nw = info.num_cores * info.num_subcores
    b_per_w = nb // nw
    idx = x.reshape(nw, b_per_w // CHUNK, CHUNK).astype(jnp.int32)
    out = _make_gather(nb, D_MODEL)(fused, idx)
    return out.reshape(b, s, D_MODEL)


# speedup vs baseline: 28.9040x; 1.4621x over previous
import functools

import jax
import jax.numpy as jnp
from jax import lax
from jax.experimental import pallas as pl
from jax.experimental.pallas import tpu as pltpu
from jax.experimental.pallas import tpu_sc as plsc

D_MODEL = 128
CHUNK = 128


def _fuse_body(emb_ref, pos_ref, out_ref):
    out_ref[...] = emb_ref[...] + pos_ref[...]


def _fuse_tables(emb_head, pos_table):
    return pl.pallas_call(
        _fuse_body,
        out_shape=jax.ShapeDtypeStruct(pos_table.shape, jnp.float32),
    )(emb_head, pos_table)


NBUF = 4
TABLE_ROWS = 512


@functools.lru_cache(maxsize=None)
def _make_gather(nb, d):
    info = plsc.get_sparse_core_info()
    nc, ns = info.num_cores, info.num_subcores
    nw = nc * ns
    b_per_w = nb // nw
    n_chunks = b_per_w // CHUNK
    n_rounds = n_chunks // NBUF
    mesh = plsc.VectorSubcoreMesh(core_axis_name="c", subcore_axis_name="s")

    @functools.partial(
        pl.kernel,
        mesh=mesh,
        out_type=jax.ShapeDtypeStruct((nb, d), jnp.float32),
        scratch_types=[
            pltpu.VMEM_SHARED((TABLE_ROWS, d), jnp.float32),
            pltpu.VMEM((n_chunks, CHUNK), jnp.int32),
            pltpu.VMEM((NBUF, CHUNK, d), jnp.float32),
        ]
        + [pltpu.SemaphoreType.DMA] * (2 * NBUF),
    )
    def gather(table_hbm, idx_hbm, out_hbm, table_s, idx_v, rows_v, *sems):
        gsems, ssems = sems[:NBUF], sems[NBUF:]
        sid = lax.axis_index("s")
        wid = sid * nc + lax.axis_index("c")

        @pl.when(sid == 0)
        def _():
            pltpu.sync_copy(table_hbm, table_s)

        pltpu.sync_copy(idx_hbm.at[wid], idx_v)
        plsc.subcore_barrier()
        base = wid * b_per_w

        def g_copy(b, g):
            return pltpu.make_async_copy(
                table_s.at[idx_v.at[g]], rows_v.at[b], gsems[b]
            )

        def s_copy(b, g):
            return pltpu.make_async_copy(
                rows_v.at[b], out_hbm.at[pl.ds(base + g * CHUNK, CHUNK)], ssems[b]
            )

        def body(t, carry):
            g0 = t * NBUF
            for b in range(NBUF):
                @pl.when(t > 0)
                def _(b=b, g0=g0):
                    s_copy(b, g0 + b - NBUF).wait()

                g_copy(b, g0 + b).start()
            for b in range(NBUF):
                g_copy(b, g0 + b).wait()
                s_copy(b, g0 + b).start()
            return carry

        lax.fori_loop(0, n_rounds, body, 0)
        for b in range(NBUF):
            s_copy(b, n_chunks - NBUF + b).wait()

    return gather


def kernel(x, emb_table, pos_table):
    b, s = x.shape
    nb = b * s
    t_rows = pos_table.shape[0]
    fused = _fuse_tables(emb_table[:t_rows], pos_table)
    info = plsc.get_sparse_core_info()
    nw = info.num_cores * info.num_subcores
    b_per_w = nb // nw
    idx = x.reshape(nw, b_per_w // CHUNK, CHUNK).astype(jnp.int32)
    out = _make_gather(nb, D_MODEL)(fused, idx)
    return out.reshape(b, s, D_MODEL)


# speedup vs baseline: 29.4042x; 1.0173x over previous
import functools

import jax
import jax.numpy as jnp
from jax import lax
from jax.experimental import pallas as pl
from jax.experimental.pallas import tpu as pltpu
from jax.experimental.pallas import tpu_sc as plsc

D_MODEL = 128
CHUNK = 128


def _fuse_body(emb_ref, pos_ref, out_ref):
    out_ref[...] = emb_ref[...] + pos_ref[...]


def _fuse_tables(emb_head, pos_table):
    return pl.pallas_call(
        _fuse_body,
        out_shape=jax.ShapeDtypeStruct(pos_table.shape, jnp.float32),
    )(emb_head, pos_table)


NBUF = 6
TABLE_ROWS = 512


@functools.lru_cache(maxsize=None)
def _make_gather(nb, d):
    info = plsc.get_sparse_core_info()
    nc, ns = info.num_cores, info.num_subcores
    nw = nc * ns
    b_per_w = nb // nw
    n_chunks = b_per_w // CHUNK
    n_rounds = n_chunks // NBUF
    mesh = plsc.VectorSubcoreMesh(core_axis_name="c", subcore_axis_name="s")

    @functools.partial(
        pl.kernel,
        mesh=mesh,
        out_type=jax.ShapeDtypeStruct((nb, d), jnp.float32),
        scratch_types=[
            pltpu.VMEM_SHARED((TABLE_ROWS, d), jnp.float32),
            pltpu.VMEM((n_chunks, CHUNK), jnp.int32),
            pltpu.VMEM((NBUF, CHUNK, d), jnp.float32),
        ]
        + [pltpu.SemaphoreType.DMA] * (2 * NBUF),
    )
    def gather(table_hbm, idx_hbm, out_hbm, table_s, idx_v, rows_v, *sems):
        gsems, ssems = sems[:NBUF], sems[NBUF:]
        sid = lax.axis_index("s")
        wid = sid * nc + lax.axis_index("c")

        @pl.when(sid == 0)
        def _():
            pltpu.sync_copy(table_hbm, table_s)

        pltpu.sync_copy(idx_hbm.at[wid], idx_v)
        plsc.subcore_barrier()
        base = wid * b_per_w

        def g_copy(b, g):
            return pltpu.make_async_copy(
                table_s.at[idx_v.at[g]], rows_v.at[b], gsems[b]
            )

        def s_copy(b, g):
            return pltpu.make_async_copy(
                rows_v.at[b], out_hbm.at[pl.ds(base + g * CHUNK, CHUNK)], ssems[b]
            )

        def body(t, carry):
            g0 = t * NBUF
            for b in range(NBUF):
                @pl.when(t > 0)
                def _(b=b, g0=g0):
                    s_copy(b, g0 + b - NBUF).wait()

                g_copy(b, g0 + b).start()
            for b in range(NBUF):
                g_copy(b, g0 + b).wait()
                s_copy(b, g0 + b).start()
            return carry

        lax.fori_loop(0, n_rounds, body, 0)
        for b in range(NBUF):
            s_copy(b, n_chunks - NBUF + b).wait()

    return gather


def kernel(x, emb_table, pos_table):
    b, s = x.shape
    nb = b * s
    t_rows = pos_table.shape[0]
    fused = _fuse_tables(emb_table[:t_rows], pos_table)
    info = plsc.get_sparse_core_info()
    nw = info.num_cores * info.num_subcores
    b_per_w = nb // nw
    idx = x.reshape(nw, b_per_w // CHUNK, CHUNK).astype(jnp.int32)
    out = _make_gather(nb, D_MODEL)(fused, idx)
    return out.reshape(b, s, D_MODEL)
